# baseline (speedup 1.0000x reference)
import functools

import jax
import jax.numpy as jnp
from jax import lax
from jax.experimental import pallas as pl
from jax.experimental.pallas import tpu as pltpu
from jax.experimental.pallas import tpu_sc as plsc

INPUT_DIM = 4096
NUM_EXPERTS = 64
TOP_K = 8
HIDDEN = 128
LN_EPS = 1e-5

BLOCK_T = 1024
NUM_WORKERS = 32
LANES = 16


def _mlp_body(x_ref, w1_ref, b1_ref, g_ref, be_ref, w2_ref, b2t_ref, pt_ref):
    h = jnp.dot(x_ref[...], w1_ref[...], preferred_element_type=jnp.float32)
    h = h + b1_ref[...]
    mean = jnp.mean(h, axis=-1, keepdims=True)
    var = jnp.mean(jnp.square(h - mean), axis=-1, keepdims=True)
    h = (h - mean) * jax.lax.rsqrt(var + LN_EPS) * g_ref[...] + be_ref[...]
    h = jnp.tanh(h)
    lt = jax.lax.dot_general(w2_ref[...], h, (((0,), (1,)), ((), ())),
                             preferred_element_type=jnp.float32)
    lt = lt + b2t_ref[...]
    m = jnp.max(lt, axis=0, keepdims=True)
    e = jnp.exp(lt - m)
    pt_ref[...] = e / jnp.sum(e, axis=0, keepdims=True)


def _probs_t(x, W1, b1, ln_gamma, ln_beta, W2, b2):
    tokens = x.shape[0]
    grid = (tokens // BLOCK_T,)
    b1 = b1.reshape(1, HIDDEN)
    ln_gamma = ln_gamma.reshape(1, HIDDEN)
    ln_beta = ln_beta.reshape(1, HIDDEN)
    b2t = b2.reshape(NUM_EXPERTS, 1)
    return pl.pallas_call(
        _mlp_body,
        grid=grid,
        in_specs=[
            pl.BlockSpec((BLOCK_T, INPUT_DIM), lambda i: (i, 0)),
            pl.BlockSpec((INPUT_DIM, HIDDEN), lambda i: (0, 0)),
            pl.BlockSpec((1, HIDDEN), lambda i: (0, 0)),
            pl.BlockSpec((1, HIDDEN), lambda i: (0, 0)),
            pl.BlockSpec((1, HIDDEN), lambda i: (0, 0)),
            pl.BlockSpec((HIDDEN, NUM_EXPERTS), lambda i: (0, 0)),
            pl.BlockSpec((NUM_EXPERTS, 1), lambda i: (0, 0)),
        ],
        out_specs=pl.BlockSpec((NUM_EXPERTS, BLOCK_T), lambda i: (0, i)),
        out_shape=jax.ShapeDtypeStruct((NUM_EXPERTS, tokens), jnp.float32),
    )(x, W1, b1, ln_gamma, ln_beta, W2, b2t)


def _make_topk_sc(tokens):
    tpw = tokens // NUM_WORKERS
    groups = tpw // LANES
    mesh = plsc.VectorSubcoreMesh(core_axis_name="c", subcore_axis_name="s",
                                  num_cores=2, num_subcores=16)

    @functools.partial(
        pl.kernel,
        out_type=(jax.ShapeDtypeStruct((TOP_K, tokens), jnp.int32),
                  jax.ShapeDtypeStruct((TOP_K, tokens), jnp.float32)),
        mesh=mesh,
        scratch_types=[
            pltpu.VMEM((NUM_EXPERTS, tpw), jnp.float32),
            pltpu.VMEM((TOP_K, tpw), jnp.int32),
            pltpu.VMEM((TOP_K, tpw), jnp.float32),
        ],
    )
    def topk_sc(pt_hbm, idx_hbm, val_hbm, pt_v, idx_v, val_v):
        wid = lax.axis_index("s") * 2 + lax.axis_index("c")
        base = wid * tpw
        pltpu.sync_copy(pt_hbm.at[:, pl.ds(base, tpw)], pt_v)

        def group_body(g, carry):
            del carry
            off = g * LANES
            cur_v = [jnp.full((LANES,), -1.0, jnp.float32) for _ in range(TOP_K)]
            cur_i = [jnp.zeros((LANES,), jnp.int32) for _ in range(TOP_K)]
            for e in range(NUM_EXPERTS):
                v = pt_v[e, pl.ds(off, LANES)]
                i = jnp.full((LANES,), e, jnp.int32)
                for j in range(TOP_K):
                    gt = v > cur_v[j]
                    cur_v[j], v = (jnp.where(gt, v, cur_v[j]),
                                   jnp.where(gt, cur_v[j], v))
                    cur_i[j], i = (jnp.where(gt, i, cur_i[j]),
                                   jnp.where(gt, cur_i[j], i))
            for j in range(TOP_K):
                idx_v[j, pl.ds(off, LANES)] = cur_i[j]
                val_v[j, pl.ds(off, LANES)] = cur_v[j]
            return 0

        lax.fori_loop(0, groups, group_body, 0)
        pltpu.sync_copy(idx_v, idx_hbm.at[:, pl.ds(base, tpw)])
        pltpu.sync_copy(val_v, val_hbm.at[:, pl.ds(base, tpw)])

    return topk_sc


@jax.jit
def kernel(x, W1, b1, ln_gamma, ln_beta, W2, b2):
    tokens = x.shape[0]
    pt = _probs_t(x, W1, b1, ln_gamma, ln_beta, W2, b2)
    idx_t, val_t = _make_topk_sc(tokens)(pt)
    return idx_t.T, val_t.T


# speedup vs baseline: 1.0056x; 1.0056x over previous
import functools

import jax
import jax.numpy as jnp
from jax import lax
from jax.experimental import pallas as pl
from jax.experimental.pallas import tpu as pltpu
from jax.experimental.pallas import tpu_sc as plsc

INPUT_DIM = 4096
NUM_EXPERTS = 64
TOP_K = 8
HIDDEN = 128
LN_EPS = 1e-5

BLOCK_T = 1024
CHUNKS = 4
NUM_WORKERS = 32
LANES = 16


def _mlp_body(x_ref, w1_ref, b1_ref, g_ref, be_ref, w2_ref, b2t_ref, pt_ref):
    h = jnp.dot(x_ref[...], w1_ref[...], preferred_element_type=jnp.float32)
    h = h + b1_ref[...]
    mean = jnp.mean(h, axis=-1, keepdims=True)
    var = jnp.mean(jnp.square(h - mean), axis=-1, keepdims=True)
    h = (h - mean) * jax.lax.rsqrt(var + LN_EPS) * g_ref[...] + be_ref[...]
    h = jnp.tanh(h)
    lt = jax.lax.dot_general(w2_ref[...], h, (((0,), (1,)), ((), ())),
                             preferred_element_type=jnp.float32)
    lt = lt + b2t_ref[...]
    m = jnp.max(lt, axis=0, keepdims=True)
    e = jnp.exp(lt - m)
    pt_ref[...] = e / jnp.sum(e, axis=0, keepdims=True)


def _probs_t_chunk(chunk, chunk_tokens, x, W1, b1, ln_gamma, ln_beta, W2, b2t):
    blocks = chunk_tokens // BLOCK_T
    base = chunk * blocks
    return pl.pallas_call(
        _mlp_body,
        grid=(blocks,),
        in_specs=[
            pl.BlockSpec((BLOCK_T, INPUT_DIM), lambda i: (base + i, 0)),
            pl.BlockSpec((INPUT_DIM, HIDDEN), lambda i: (0, 0)),
            pl.BlockSpec((1, HIDDEN), lambda i: (0, 0)),
            pl.BlockSpec((1, HIDDEN), lambda i: (0, 0)),
            pl.BlockSpec((1, HIDDEN), lambda i: (0, 0)),
            pl.BlockSpec((HIDDEN, NUM_EXPERTS), lambda i: (0, 0)),
            pl.BlockSpec((NUM_EXPERTS, 1), lambda i: (0, 0)),
        ],
        out_specs=pl.BlockSpec((NUM_EXPERTS, BLOCK_T), lambda i: (0, i)),
        out_shape=jax.ShapeDtypeStruct((NUM_EXPERTS, chunk_tokens),
                                       jnp.float32),
    )(x, W1, b1, ln_gamma, ln_beta, W2, b2t)


def _make_topk_sc(chunk_tokens):
    tpw = chunk_tokens // NUM_WORKERS
    groups = tpw // LANES
    mesh = plsc.VectorSubcoreMesh(core_axis_name="c", subcore_axis_name="s",
                                  num_cores=2, num_subcores=16)

    @functools.partial(
        pl.kernel,
        out_type=(jax.ShapeDtypeStruct((TOP_K, chunk_tokens), jnp.int32),
                  jax.ShapeDtypeStruct((TOP_K, chunk_tokens), jnp.float32)),
        mesh=mesh,
        scratch_types=[
            pltpu.VMEM((NUM_EXPERTS, tpw), jnp.float32),
            pltpu.VMEM((TOP_K, tpw), jnp.int32),
            pltpu.VMEM((TOP_K, tpw), jnp.float32),
        ],
    )
    def topk_sc(pt_hbm, idx_hbm, val_hbm, pt_v, idx_v, val_v):
        wid = lax.axis_index("s") * 2 + lax.axis_index("c")
        base = wid * tpw
        pltpu.sync_copy(pt_hbm.at[:, pl.ds(base, tpw)], pt_v)

        def group_body(g, carry):
            del carry
            off = g * LANES
            cur_v = [jnp.full((LANES,), -1.0, jnp.float32)
                     for _ in range(TOP_K)]
            cur_i = [jnp.zeros((LANES,), jnp.int32) for _ in range(TOP_K)]
            for e in range(NUM_EXPERTS):
                v = pt_v[e, pl.ds(off, LANES)]
                i = jnp.full((LANES,), e, jnp.int32)
                for j in range(TOP_K):
                    gt = v > cur_v[j]
                    cur_v[j], v = (jnp.where(gt, v, cur_v[j]),
                                   jnp.where(gt, cur_v[j], v))
                    cur_i[j], i = (jnp.where(gt, i, cur_i[j]),
                                   jnp.where(gt, cur_i[j], i))
            for j in range(TOP_K):
                idx_v[j, pl.ds(off, LANES)] = cur_i[j]
                val_v[j, pl.ds(off, LANES)] = cur_v[j]
            return 0

        lax.fori_loop(0, groups, group_body, 0)
        pltpu.sync_copy(idx_v, idx_hbm.at[:, pl.ds(base, tpw)])
        pltpu.sync_copy(val_v, val_hbm.at[:, pl.ds(base, tpw)])

    return topk_sc


@jax.jit
def kernel(x, W1, b1, ln_gamma, ln_beta, W2, b2):
    tokens = x.shape[0]
    chunk_tokens = tokens // CHUNKS
    b1 = b1.reshape(1, HIDDEN)
    ln_gamma = ln_gamma.reshape(1, HIDDEN)
    ln_beta = ln_beta.reshape(1, HIDDEN)
    b2t = b2.reshape(NUM_EXPERTS, 1)
    topk_sc = _make_topk_sc(chunk_tokens)
    idx_parts = []
    val_parts = []
    for c in range(CHUNKS):
        pt = _probs_t_chunk(c, chunk_tokens, x, W1, b1, ln_gamma, ln_beta,
                            W2, b2t)
        idx_t, val_t = topk_sc(pt)
        idx_parts.append(idx_t)
        val_parts.append(val_t)
    idx = jnp.concatenate(idx_parts, axis=1).T
    vals = jnp.concatenate(val_parts, axis=1).T
    return idx, vals


# speedup vs baseline: 1.0131x; 1.0075x over previous
import functools

import jax
import jax.numpy as jnp
from jax import lax
from jax.experimental import pallas as pl
from jax.experimental.pallas import tpu as pltpu
from jax.experimental.pallas import tpu_sc as plsc

INPUT_DIM = 4096
NUM_EXPERTS = 64
TOP_K = 8
HIDDEN = 128
LN_EPS = 1e-5

BLOCK_T = 1024
CHUNK_SIZES = (12288, 12288, 4096, 4096)
NUM_WORKERS = 32
LANES = 16


def _mlp_body(x_ref, w1_ref, b1_ref, g_ref, be_ref, w2_ref, b2t_ref, pt_ref):
    h = jnp.dot(x_ref[...], w1_ref[...], preferred_element_type=jnp.float32)
    h = h + b1_ref[...]
    mean = jnp.mean(h, axis=-1, keepdims=True)
    var = jnp.mean(jnp.square(h - mean), axis=-1, keepdims=True)
    h = (h - mean) * jax.lax.rsqrt(var + LN_EPS) * g_ref[...] + be_ref[...]
    h = jnp.tanh(h)
    lt = jax.lax.dot_general(w2_ref[...], h, (((0,), (1,)), ((), ())),
                             preferred_element_type=jnp.float32)
    lt = lt + b2t_ref[...]
    m = jnp.max(lt, axis=0, keepdims=True)
    e = jnp.exp(lt - m)
    pt_ref[...] = e / jnp.sum(e, axis=0, keepdims=True)


def _probs_t_chunk(base_block, chunk_tokens, x, W1, b1, ln_gamma, ln_beta,
                   W2, b2t):
    blocks = chunk_tokens // BLOCK_T
    base = base_block
    return pl.pallas_call(
        _mlp_body,
        grid=(blocks,),
        in_specs=[
            pl.BlockSpec((BLOCK_T, INPUT_DIM), lambda i: (base + i, 0)),
            pl.BlockSpec((INPUT_DIM, HIDDEN), lambda i: (0, 0)),
            pl.BlockSpec((1, HIDDEN), lambda i: (0, 0)),
            pl.BlockSpec((1, HIDDEN), lambda i: (0, 0)),
            pl.BlockSpec((1, HIDDEN), lambda i: (0, 0)),
            pl.BlockSpec((HIDDEN, NUM_EXPERTS), lambda i: (0, 0)),
            pl.BlockSpec((NUM_EXPERTS, 1), lambda i: (0, 0)),
        ],
        out_specs=pl.BlockSpec((NUM_EXPERTS, BLOCK_T), lambda i: (0, i)),
        out_shape=jax.ShapeDtypeStruct((NUM_EXPERTS, chunk_tokens),
                                       jnp.float32),
    )(x, W1, b1, ln_gamma, ln_beta, W2, b2t)


def _make_topk_sc(chunk_tokens):
    tpw = chunk_tokens // NUM_WORKERS
    groups = tpw // LANES
    mesh = plsc.VectorSubcoreMesh(core_axis_name="c", subcore_axis_name="s",
                                  num_cores=2, num_subcores=16)

    @functools.partial(
        pl.kernel,
        out_type=(jax.ShapeDtypeStruct((TOP_K, chunk_tokens), jnp.int32),
                  jax.ShapeDtypeStruct((TOP_K, chunk_tokens), jnp.float32)),
        mesh=mesh,
        scratch_types=[
            pltpu.VMEM((NUM_EXPERTS, tpw), jnp.float32),
            pltpu.VMEM((TOP_K, tpw), jnp.int32),
            pltpu.VMEM((TOP_K, tpw), jnp.float32),
        ],
    )
    def topk_sc(pt_hbm, idx_hbm, val_hbm, pt_v, idx_v, val_v):
        wid = lax.axis_index("s") * 2 + lax.axis_index("c")
        base = wid * tpw
        pltpu.sync_copy(pt_hbm.at[:, pl.ds(base, tpw)], pt_v)

        def group_body(g, carry):
            del carry
            off = g * LANES
            cur_v = [jnp.full((LANES,), -1.0, jnp.float32)
                     for _ in range(TOP_K)]
            cur_i = [jnp.zeros((LANES,), jnp.int32) for _ in range(TOP_K)]
            for e in range(NUM_EXPERTS):
                v = pt_v[e, pl.ds(off, LANES)]
                i = jnp.full((LANES,), e, jnp.int32)
                for j in range(TOP_K):
                    gt = v > cur_v[j]
                    cur_v[j], v = (jnp.where(gt, v, cur_v[j]),
                                   jnp.where(gt, cur_v[j], v))
                    cur_i[j], i = (jnp.where(gt, i, cur_i[j]),
                                   jnp.where(gt, cur_i[j], i))
            for j in range(TOP_K):
                idx_v[j, pl.ds(off, LANES)] = cur_i[j]
                val_v[j, pl.ds(off, LANES)] = cur_v[j]
            return 0

        lax.fori_loop(0, groups, group_body, 0)
        pltpu.sync_copy(idx_v, idx_hbm.at[:, pl.ds(base, tpw)])
        pltpu.sync_copy(val_v, val_hbm.at[:, pl.ds(base, tpw)])

    return topk_sc


@jax.jit
def kernel(x, W1, b1, ln_gamma, ln_beta, W2, b2):
    b1 = b1.reshape(1, HIDDEN)
    ln_gamma = ln_gamma.reshape(1, HIDDEN)
    ln_beta = ln_beta.reshape(1, HIDDEN)
    b2t = b2.reshape(NUM_EXPERTS, 1)
    topk_sc = {n: _make_topk_sc(n) for n in set(CHUNK_SIZES)}
    idx_parts = []
    val_parts = []
    base_block = 0
    for chunk_tokens in CHUNK_SIZES:
        pt = _probs_t_chunk(base_block, chunk_tokens, x, W1, b1, ln_gamma,
                            ln_beta, W2, b2t)
        idx_c, val_c = topk_sc[chunk_tokens](pt)
        idx_parts.append(idx_c)
        val_parts.append(val_c)
        base_block += chunk_tokens // BLOCK_T
    idx = jnp.concatenate(idx_parts, axis=1).T
    vals = jnp.concatenate(val_parts, axis=1).T
    return idx, vals


# speedup vs baseline: 1.0396x; 1.0261x over previous
import functools

import jax
import jax.numpy as jnp
from jax import lax
from jax.experimental import pallas as pl
from jax.experimental.pallas import tpu as pltpu
from jax.experimental.pallas import tpu_sc as plsc

INPUT_DIM = 4096
NUM_EXPERTS = 64
TOP_K = 8
HIDDEN = 128
LN_EPS = 1e-5

BLOCK_T = 1024
CHUNK_SIZES = (16384, 12288, 4096)
NUM_WORKERS = 32
LANES = 16


def _mlp_body(x_ref, w1_ref, b1_ref, g_ref, be_ref, w2_ref, b2t_ref, pt_ref):
    h = jnp.dot(x_ref[...], w1_ref[...], preferred_element_type=jnp.float32)
    h = h + b1_ref[...]
    mean = jnp.mean(h, axis=-1, keepdims=True)
    var = jnp.mean(jnp.square(h - mean), axis=-1, keepdims=True)
    h = (h - mean) * jax.lax.rsqrt(var + LN_EPS) * g_ref[...] + be_ref[...]
    h = jnp.tanh(h)
    lt = jax.lax.dot_general(w2_ref[...], h, (((0,), (1,)), ((), ())),
                             preferred_element_type=jnp.float32)
    lt = lt + b2t_ref[...]
    m = jnp.max(lt, axis=0, keepdims=True)
    e = jnp.exp(lt - m)
    pt_ref[...] = e / jnp.sum(e, axis=0, keepdims=True)


def _probs_t_chunk(base_block, chunk_tokens, x, W1, b1, ln_gamma, ln_beta,
                   W2, b2t):
    blocks = chunk_tokens // BLOCK_T
    base = base_block
    return pl.pallas_call(
        _mlp_body,
        grid=(blocks,),
        in_specs=[
            pl.BlockSpec((BLOCK_T, INPUT_DIM), lambda i: (base + i, 0)),
            pl.BlockSpec((INPUT_DIM, HIDDEN), lambda i: (0, 0)),
            pl.BlockSpec((1, HIDDEN), lambda i: (0, 0)),
            pl.BlockSpec((1, HIDDEN), lambda i: (0, 0)),
            pl.BlockSpec((1, HIDDEN), lambda i: (0, 0)),
            pl.BlockSpec((HIDDEN, NUM_EXPERTS), lambda i: (0, 0)),
            pl.BlockSpec((NUM_EXPERTS, 1), lambda i: (0, 0)),
        ],
        out_specs=pl.BlockSpec((NUM_EXPERTS, BLOCK_T), lambda i: (0, i)),
        out_shape=jax.ShapeDtypeStruct((NUM_EXPERTS, chunk_tokens),
                                       jnp.float32),
    )(x, W1, b1, ln_gamma, ln_beta, W2, b2t)


def _make_topk_sc(chunk_tokens):
    tpw = chunk_tokens // NUM_WORKERS
    groups = tpw // LANES
    mesh = plsc.VectorSubcoreMesh(core_axis_name="c", subcore_axis_name="s",
                                  num_cores=2, num_subcores=16)

    @functools.partial(
        pl.kernel,
        out_type=(jax.ShapeDtypeStruct((TOP_K, chunk_tokens), jnp.int32),
                  jax.ShapeDtypeStruct((TOP_K, chunk_tokens), jnp.float32)),
        mesh=mesh,
        scratch_types=[
            pltpu.VMEM((NUM_EXPERTS, tpw), jnp.float32),
            pltpu.VMEM((TOP_K, tpw), jnp.int32),
            pltpu.VMEM((TOP_K, tpw), jnp.float32),
        ],
    )
    def topk_sc(pt_hbm, idx_hbm, val_hbm, pt_v, idx_v, val_v):
        wid = lax.axis_index("s") * 2 + lax.axis_index("c")
        base = wid * tpw
        pltpu.sync_copy(pt_hbm.at[:, pl.ds(base, tpw)], pt_v)

        def group_body(g, carry):
            del carry
            off = g * LANES
            cur_v = [jnp.full((LANES,), -1.0, jnp.float32)
                     for _ in range(TOP_K)]
            cur_i = [jnp.zeros((LANES,), jnp.int32) for _ in range(TOP_K)]
            for e in range(NUM_EXPERTS):
                v = pt_v[e, pl.ds(off, LANES)]
                i = jnp.full((LANES,), e, jnp.int32)
                for j in range(TOP_K):
                    gt = v > cur_v[j]
                    cur_v[j], v = (jnp.where(gt, v, cur_v[j]),
                                   jnp.where(gt, cur_v[j], v))
                    cur_i[j], i = (jnp.where(gt, i, cur_i[j]),
                                   jnp.where(gt, cur_i[j], i))
            for j in range(TOP_K):
                idx_v[j, pl.ds(off, LANES)] = cur_i[j]
                val_v[j, pl.ds(off, LANES)] = cur_v[j]
            return 0

        lax.fori_loop(0, groups, group_body, 0)
        pltpu.sync_copy(idx_v, idx_hbm.at[:, pl.ds(base, tpw)])
        pltpu.sync_copy(val_v, val_hbm.at[:, pl.ds(base, tpw)])

    return topk_sc


@jax.jit
def kernel(x, W1, b1, ln_gamma, ln_beta, W2, b2):
    b1 = b1.reshape(1, HIDDEN)
    ln_gamma = ln_gamma.reshape(1, HIDDEN)
    ln_beta = ln_beta.reshape(1, HIDDEN)
    b2t = b2.reshape(NUM_EXPERTS, 1)
    topk_sc = {n: _make_topk_sc(n) for n in set(CHUNK_SIZES)}
    idx_parts = []
    val_parts = []
    base_block = 0
    for chunk_tokens in CHUNK_SIZES:
        pt = _probs_t_chunk(base_block, chunk_tokens, x, W1, b1, ln_gamma,
                            ln_beta, W2, b2t)
        idx_c, val_c = topk_sc[chunk_tokens](pt)
        idx_parts.append(idx_c)
        val_parts.append(val_c)
        base_block += chunk_tokens // BLOCK_T
    idx = jnp.concatenate(idx_parts, axis=1).T
    vals = jnp.concatenate(val_parts, axis=1).T
    return idx, vals


# speedup vs baseline: 1.3013x; 1.2517x over previous
import functools

import jax
import jax.numpy as jnp
from jax.experimental import pallas as pl

INPUT_DIM = 4096
NUM_EXPERTS = 64
TOP_K = 8
HIDDEN = 128
LN_EPS = 1e-5

BLOCK_T = 1024


def _fused_body(x_ref, w1_ref, b1_ref, g_ref, be_ref, w2_ref, b2t_ref,
                idx_ref, val_ref):
    h = jnp.dot(x_ref[...], w1_ref[...], preferred_element_type=jnp.float32)
    h = h + b1_ref[...]
    mean = jnp.mean(h, axis=-1, keepdims=True)
    var = jnp.mean(jnp.square(h - mean), axis=-1, keepdims=True)
    h = (h - mean) * jax.lax.rsqrt(var + LN_EPS) * g_ref[...] + be_ref[...]
    h = jnp.tanh(h)
    lt = jax.lax.dot_general(w2_ref[...], h, (((0,), (1,)), ((), ())),
                             preferred_element_type=jnp.float32)
    lt = lt + b2t_ref[...]
    m = jnp.max(lt, axis=0, keepdims=True)
    e = jnp.exp(lt - m)
    probs = e / jnp.sum(e, axis=0, keepdims=True)

    eidx = jax.lax.broadcasted_iota(jnp.int32, probs.shape, 0)
    work = probs
    idx_rows = []
    val_rows = []
    for _ in range(TOP_K):
        mx = jnp.max(work, axis=0, keepdims=True)
        amx = jnp.min(jnp.where(work == mx, eidx, NUM_EXPERTS),
                      axis=0, keepdims=True)
        idx_rows.append(amx)
        val_rows.append(mx)
        work = jnp.where(eidx == amx, -1.0, work)
    idx_ref[...] = jnp.concatenate(idx_rows, axis=0)
    val_ref[...] = jnp.concatenate(val_rows, axis=0)


@functools.partial(jax.jit, static_argnames=())
def kernel(x, W1, b1, ln_gamma, ln_beta, W2, b2):
    tokens = x.shape[0]
    grid = (tokens // BLOCK_T,)
    b1 = b1.reshape(1, HIDDEN)
    ln_gamma = ln_gamma.reshape(1, HIDDEN)
    ln_beta = ln_beta.reshape(1, HIDDEN)
    b2t = b2.reshape(NUM_EXPERTS, 1)
    idx_t, val_t = pl.pallas_call(
        _fused_body,
        grid=grid,
        in_specs=[
            pl.BlockSpec((BLOCK_T, INPUT_DIM), lambda i: (i, 0)),
            pl.BlockSpec((INPUT_DIM, HIDDEN), lambda i: (0, 0)),
            pl.BlockSpec((1, HIDDEN), lambda i: (0, 0)),
            pl.BlockSpec((1, HIDDEN), lambda i: (0, 0)),
            pl.BlockSpec((1, HIDDEN), lambda i: (0, 0)),
            pl.BlockSpec((HIDDEN, NUM_EXPERTS), lambda i: (0, 0)),
            pl.BlockSpec((NUM_EXPERTS, 1), lambda i: (0, 0)),
        ],
        out_specs=[
            pl.BlockSpec((TOP_K, BLOCK_T), lambda i: (0, i)),
            pl.BlockSpec((TOP_K, BLOCK_T), lambda i: (0, i)),
        ],
        out_shape=[
            jax.ShapeDtypeStruct((TOP_K, tokens), jnp.int32),
            jax.ShapeDtypeStruct((TOP_K, tokens), jnp.float32),
        ],
    )(x, W1, b1, ln_gamma, ln_beta, W2, b2t)
    return idx_t.T, val_t.T
